# baseline (speedup 1.0000x reference)
import math

import jax
import jax.numpy as jnp
from jax.experimental import pallas as pl
from jax.experimental.pallas import tpu as pltpu

EMBED = 1024
HEADS = 16
HEAD_DIM = 64
SCALE = HEAD_DIM ** -0.5
WINDOW = 2
FAR = 4
S = 2048
BQ = 256
BK = 512

_LOG1E4 = math.log(10000.0)


def _qkv_rope_kernel(x_ref, w_ref, b_ref, o_ref):
    qi = pl.program_id(0)
    y = jnp.dot(x_ref[...], w_ref[...], preferred_element_type=jnp.float32)
    y = y + b_ref[...]
    bq, n = y.shape
    y_p32 = jnp.concatenate([y[:, 32:], y[:, :32]], axis=1)
    y_m32 = jnp.concatenate([y[:, -32:], y[:, :-32]], axis=1)
    col = jax.lax.broadcasted_iota(jnp.int32, (bq, n), 1)
    d = col % HEAD_DIM
    dr = d % (HEAD_DIM // 2)
    hi = d >= (HEAD_DIM // 2)
    partner = jnp.where(hi, y_m32, y_p32)
    inv = jnp.exp(dr.astype(jnp.float32) * (-_LOG1E4 / (HEAD_DIM // 2)))
    row = jax.lax.broadcasted_iota(jnp.int32, (bq, n), 0)
    pos = (qi * bq + row).astype(jnp.float32)
    ang = pos * inv
    c = jnp.cos(ang)
    s = jnp.sin(ang)
    roped = y * c + partner * jnp.where(hi, s, -s)
    out = jnp.where(col < 2 * EMBED, roped, y)
    out = out * jnp.where(col < EMBED, SCALE, 1.0)
    o_ref[...] = out.astype(jnp.bfloat16)


def _attn_kernel(kmax_ref, barc_ref, barr_ref, instc_ref, instr_ref, qkv_ref,
                 o_ref, bias_ref, ms_ref, mc_ref, mg_ref):
    qi = pl.program_id(0)
    hp = pl.program_id(1)
    s = qkv_ref.shape[0]
    nkb = (kmax_ref[qi] + BK - 1) // BK

    @pl.when(hp == 0)
    def _build_masks():
        bar_q = barc_ref[...]
        inst_q = instc_ref[...]

        def mbody(kb, _):
            j0 = kb * BK
            i = qi * BQ + jax.lax.broadcasted_iota(jnp.int32, (BQ, BK), 0)
            j = j0 + jax.lax.broadcasted_iota(jnp.int32, (BQ, BK), 1)
            causal = j <= i
            bar_k = barr_ref[0:1, pl.ds(j0, BK)]
            inst_k = instr_ref[0:1, pl.ds(j0, BK)]
            same = (inst_q == inst_k) & (inst_q < 129) & causal
            off = bar_q - bar_k
            nearfar = ((off >= 0) & (off <= WINDOW)) | (off == FAR)
            cross = ((inst_q < 129) & (bar_q >= 0) & (inst_k != inst_q)
                     & (inst_k < 129) & nearfar)
            glob = ((inst_k == 129) | (bar_k == -1)) & causal
            union = same | cross | glob
            ms_ref[:, pl.ds(j0, BK)] = same.astype(jnp.float32)
            mc_ref[:, pl.ds(j0, BK)] = cross.astype(jnp.float32)
            mg_ref[:, pl.ds(j0, BK)] = glob.astype(jnp.float32)
            bias_ref[:, pl.ds(j0, BK)] = jnp.where(union, 0.0, -1e30)
            return 0

        jax.lax.fori_loop(0, nkb, mbody, 0)

    q2 = qkv_ref[pl.ds(qi * BQ, BQ), pl.ds(hp * 128, 128)]
    qa = q2[:, :HEAD_DIM].astype(jnp.float32)
    qb = q2[:, HEAD_DIM:].astype(jnp.float32)
    k2all = qkv_ref[:, pl.ds(EMBED + hp * 128, 128)].astype(jnp.float32)
    kn2a = jnp.sum(k2all[:, :HEAD_DIM] ** 2, axis=1, keepdims=True)
    kn2b = jnp.sum(k2all[:, HEAD_DIM:] ** 2, axis=1, keepdims=True)
    ma = jnp.sqrt(jnp.sum(qa * qa, axis=1, keepdims=True) * jnp.max(kn2a))
    mb = jnp.sqrt(jnp.sum(qb * qb, axis=1, keepdims=True) * jnp.max(kn2b))
    qa16 = q2[:, :HEAD_DIM]
    qb16 = q2[:, HEAD_DIM:]

    zs = jnp.zeros((BQ, 1), jnp.float32)
    za = jnp.zeros((BQ, HEAD_DIM), jnp.float32)
    carry = (zs, zs, zs, za, za, za, zs, zs, zs, za, za, za)

    def body(kb, c):
        j0 = kb * BK
        kblk = qkv_ref[pl.ds(j0, BK), pl.ds(EMBED + hp * 128, 128)]
        vblk = qkv_ref[pl.ds(j0, BK), pl.ds(2 * EMBED + hp * 128, 128)]
        bias = bias_ref[:, pl.ds(j0, BK)]
        msb = ms_ref[:, pl.ds(j0, BK)]
        mcb = mc_ref[:, pl.ds(j0, BK)]
        mgb = mg_ref[:, pl.ds(j0, BK)]
        out = []
        for (q16, mrow, t, off0) in ((qa16, ma, 0, 0), (qb16, mb, 1, 6)):
            k = kblk[:, t * HEAD_DIM:(t + 1) * HEAD_DIM]
            v = vblk[:, t * HEAD_DIM:(t + 1) * HEAD_DIM]
            sc = jax.lax.dot_general(
                q16, k, (((1,), (1,)), ((), ())),
                preferred_element_type=jnp.float32)
            e = jnp.exp(sc + bias - mrow)
            s1, s2, s3, a1, a2, a3 = c[off0:off0 + 6]
            for mask, spos, apos in ((msb, 0, 3), (mcb, 1, 4), (mgb, 2, 5)):
                em = e * mask
                ss = c[off0 + spos] + jnp.sum(em, axis=1, keepdims=True)
                aa = c[off0 + apos] + jnp.dot(
                    em.astype(jnp.bfloat16), v,
                    preferred_element_type=jnp.float32)
                out.append((off0 + spos, ss))
                out.append((off0 + apos, aa))
        c = list(c)
        for idx, val in out:
            c[idx] = val
        return tuple(c)

    res = jax.lax.fori_loop(0, nkb, body, carry)
    halves = []
    for off0 in (0, 6):
        s1, s2, s3, a1, a2, a3 = res[off0:off0 + 6]
        o = (a1 / jnp.where(s1 == 0.0, 1.0, s1)
             + a2 / jnp.where(s2 == 0.0, 1.0, s2)
             + a3 / jnp.where(s3 == 0.0, 1.0, s3))
        halves.append(o)
    o_ref[...] = jnp.concatenate(halves, axis=1).astype(jnp.bfloat16)


def _out_proj_kernel(a_ref, w_ref, b_ref, o_ref):
    o_ref[...] = jnp.dot(a_ref[...], w_ref[...],
                         preferred_element_type=jnp.float32) + b_ref[...]


@jax.jit
def kernel(x, bar_ids, instrument_ids, Wq, bq, Wk, bk, Wv, bv, Wo, bo):
    B, s, e = x.shape
    x2 = x.reshape(s, e).astype(jnp.bfloat16)
    Wqkv = jnp.concatenate([Wq.T, Wk.T, Wv.T], axis=1).astype(jnp.bfloat16)
    bqkv = jnp.concatenate([bq, bk, bv]).reshape(1, 3 * e)

    nq = s // BQ
    qkv = pl.pallas_call(
        _qkv_rope_kernel,
        grid=(nq,),
        in_specs=[
            pl.BlockSpec((BQ, e), lambda i: (i, 0)),
            pl.BlockSpec((e, 3 * e), lambda i: (0, 0)),
            pl.BlockSpec((1, 3 * e), lambda i: (0, 0)),
        ],
        out_specs=pl.BlockSpec((BQ, 3 * e), lambda i: (i, 0)),
        out_shape=jax.ShapeDtypeStruct((s, 3 * e), jnp.bfloat16),
    )(x2, Wqkv, bqkv)

    bar = bar_ids.reshape(s)
    bar_c = bar_ids.reshape(s, 1)
    bar_r = bar_ids.reshape(1, s)
    inst_c = instrument_ids.reshape(s, 1)
    inst_r = instrument_ids.reshape(1, s)

    blk_last = bar[BQ - 1::BQ]
    cross_end = jnp.searchsorted(bar, blk_last, side='right').astype(jnp.int32)
    causal_end = (jnp.arange(1, nq + 1, dtype=jnp.int32)) * BQ
    kmax = jnp.maximum(cross_end, causal_end)

    attn = pl.pallas_call(
        _attn_kernel,
        grid=(nq, HEADS // 2),
        in_specs=[
            pl.BlockSpec(memory_space=pltpu.SMEM),
            pl.BlockSpec((BQ, 1), lambda i, h: (i, 0)),
            pl.BlockSpec((1, S), lambda i, h: (0, 0)),
            pl.BlockSpec((BQ, 1), lambda i, h: (i, 0)),
            pl.BlockSpec((1, S), lambda i, h: (0, 0)),
            pl.BlockSpec((S, 3 * EMBED), lambda i, h: (0, 0)),
        ],
        out_specs=pl.BlockSpec((BQ, 2 * HEAD_DIM), lambda i, h: (i, h)),
        out_shape=jax.ShapeDtypeStruct((s, e), jnp.bfloat16),
        scratch_shapes=[
            pltpu.VMEM((BQ, S), jnp.float32),
            pltpu.VMEM((BQ, S), jnp.float32),
            pltpu.VMEM((BQ, S), jnp.float32),
            pltpu.VMEM((BQ, S), jnp.float32),
        ],
    )(kmax, bar_c, bar_r, inst_c, inst_r, qkv)

    out = pl.pallas_call(
        _out_proj_kernel,
        grid=(nq,),
        in_specs=[
            pl.BlockSpec((BQ, e), lambda i: (i, 0)),
            pl.BlockSpec((e, e), lambda i: (0, 0)),
            pl.BlockSpec((1, e), lambda i: (0, 0)),
        ],
        out_specs=pl.BlockSpec((BQ, e), lambda i: (i, 0)),
        out_shape=jax.ShapeDtypeStruct((s, e), jnp.float32),
    )(attn, Wo.T.astype(jnp.bfloat16), bo.reshape(1, e))

    return out.reshape(B, s, e)


# speedup vs baseline: 1.0692x; 1.0692x over previous
import math

import jax
import jax.numpy as jnp
from jax.experimental import pallas as pl
from jax.experimental.pallas import tpu as pltpu

EMBED = 1024
HEADS = 16
HEAD_DIM = 64
SCALE = HEAD_DIM ** -0.5
WINDOW = 2
FAR = 4
S = 2048
BQ = 256
BK = 512

_LOG1E4 = math.log(10000.0)


def _qkv_rope_kernel(x_ref, w_ref, b_ref, o_ref):
    qi = pl.program_id(0)
    y = jnp.dot(x_ref[...], w_ref[...], preferred_element_type=jnp.float32)
    y = y + b_ref[...]
    bq, n = y.shape
    y_p32 = jnp.concatenate([y[:, 32:], y[:, :32]], axis=1)
    y_m32 = jnp.concatenate([y[:, -32:], y[:, :-32]], axis=1)
    col = jax.lax.broadcasted_iota(jnp.int32, (bq, n), 1)
    d = col % HEAD_DIM
    dr = d % (HEAD_DIM // 2)
    hi = d >= (HEAD_DIM // 2)
    partner = jnp.where(hi, y_m32, y_p32)
    inv = jnp.exp(dr.astype(jnp.float32) * (-_LOG1E4 / (HEAD_DIM // 2)))
    row = jax.lax.broadcasted_iota(jnp.int32, (bq, n), 0)
    pos = (qi * bq + row).astype(jnp.float32)
    ang = pos * inv
    c = jnp.cos(ang)
    s = jnp.sin(ang)
    roped = y * c + partner * jnp.where(hi, s, -s)
    out = jnp.where(col < 2 * EMBED, roped, y)
    out = out * jnp.where(col < EMBED, SCALE, 1.0)
    o_ref[...] = out.astype(jnp.bfloat16)


def _attn_kernel(kmax_ref, barc_ref, barr_ref, instc_ref, instr_ref, qkv_ref,
                 o_ref, bias_ref, ms_ref, mc_ref, mg_ref, acc_refs, sum_refs):
    qi = pl.program_id(0)
    hp = pl.program_id(1)
    kmax = kmax_ref[qi]
    nkb = S // BK

    @pl.when(hp == 0)
    def _build_masks():
        bar_q = barc_ref[...]
        inst_q = instc_ref[...]
        for kb in range(nkb):
            @pl.when(kb * BK < kmax)
            def _mblk(kb=kb):
                j0 = kb * BK
                i = qi * BQ + jax.lax.broadcasted_iota(jnp.int32, (BQ, BK), 0)
                j = j0 + jax.lax.broadcasted_iota(jnp.int32, (BQ, BK), 1)
                causal = j <= i
                bar_k = barr_ref[0:1, j0:j0 + BK]
                inst_k = instr_ref[0:1, j0:j0 + BK]
                same = (inst_q == inst_k) & (inst_q < 129) & causal
                off = bar_q - bar_k
                nearfar = ((off >= 0) & (off <= WINDOW)) | (off == FAR)
                cross = ((inst_q < 129) & (bar_q >= 0) & (inst_k != inst_q)
                         & (inst_k < 129) & nearfar)
                glob = ((inst_k == 129) | (bar_k == -1)) & causal
                union = same | cross | glob
                ms_ref[:, j0:j0 + BK] = same.astype(jnp.float32)
                mc_ref[:, j0:j0 + BK] = cross.astype(jnp.float32)
                mg_ref[:, j0:j0 + BK] = glob.astype(jnp.float32)
                bias_ref[:, j0:j0 + BK] = jnp.where(union, 0.0, -1e30)

    q2 = qkv_ref[pl.ds(qi * BQ, BQ), pl.ds(hp * 128, 128)]
    qa = q2[:, :HEAD_DIM].astype(jnp.float32)
    qb = q2[:, HEAD_DIM:].astype(jnp.float32)
    k2all = qkv_ref[:, pl.ds(EMBED + hp * 128, 128)].astype(jnp.float32)
    kn2a = jnp.sum(k2all[:, :HEAD_DIM] ** 2, axis=1, keepdims=True)
    kn2b = jnp.sum(k2all[:, HEAD_DIM:] ** 2, axis=1, keepdims=True)
    ma = jnp.sqrt(jnp.sum(qa * qa, axis=1, keepdims=True) * jnp.max(kn2a))
    mb = jnp.sqrt(jnp.sum(qb * qb, axis=1, keepdims=True) * jnp.max(kn2b))
    shift = (ma, mb)
    q16 = (q2[:, :HEAD_DIM], q2[:, HEAD_DIM:])

    for r in sum_refs:
        r[...] = jnp.zeros_like(r)

    for kb in range(nkb):
        @pl.when(kb * BK < kmax)
        def _kblk(kb=kb):
            j0 = kb * BK
            kblk = qkv_ref[pl.ds(j0, BK), pl.ds(EMBED + hp * 128, 128)]
            vblk = qkv_ref[pl.ds(j0, BK), pl.ds(2 * EMBED + hp * 128, 128)]
            bias = bias_ref[:, j0:j0 + BK]
            masks = (ms_ref[:, j0:j0 + BK], mc_ref[:, j0:j0 + BK],
                     mg_ref[:, j0:j0 + BK])
            for t in range(2):
                k = kblk[:, t * HEAD_DIM:(t + 1) * HEAD_DIM]
                v = vblk[:, t * HEAD_DIM:(t + 1) * HEAD_DIM]
                sc = jax.lax.dot_general(
                    q16[t], k, (((1,), (1,)), ((), ())),
                    preferred_element_type=jnp.float32)
                e = jnp.exp(sc + bias - shift[t])
                for c in range(3):
                    em = e * masks[c]
                    sum_refs[3 * t + c][:, kb:kb + 1] = jnp.sum(
                        em, axis=1, keepdims=True)
                    pa = jnp.dot(em.astype(jnp.bfloat16), v,
                                 preferred_element_type=jnp.float32)
                    col = t * HEAD_DIM
                    if kb == 0:
                        acc_refs[c][:, col:col + HEAD_DIM] = pa
                    else:
                        acc_refs[c][:, col:col + HEAD_DIM] = (
                            acc_refs[c][:, col:col + HEAD_DIM] + pa)

    halves = []
    for t in range(2):
        o = None
        for c in range(3):
            ssum = jnp.sum(sum_refs[3 * t + c][...], axis=1, keepdims=True)
            a = acc_refs[c][:, t * HEAD_DIM:(t + 1) * HEAD_DIM]
            part = a / jnp.where(ssum == 0.0, 1.0, ssum)
            o = part if o is None else o + part
        halves.append(o)
    o_ref[...] = jnp.concatenate(halves, axis=1).astype(jnp.bfloat16)


def _out_proj_kernel(a_ref, w_ref, b_ref, o_ref):
    o_ref[...] = jnp.dot(a_ref[...], w_ref[...],
                         preferred_element_type=jnp.float32) + b_ref[...]


@jax.jit
def kernel(x, bar_ids, instrument_ids, Wq, bq, Wk, bk, Wv, bv, Wo, bo):
    B, s, e = x.shape
    x2 = x.reshape(s, e).astype(jnp.bfloat16)
    Wqkv = jnp.concatenate([Wq.T, Wk.T, Wv.T], axis=1).astype(jnp.bfloat16)
    bqkv = jnp.concatenate([bq, bk, bv]).reshape(1, 3 * e)

    nq = s // BQ
    qkv = pl.pallas_call(
        _qkv_rope_kernel,
        grid=(nq,),
        in_specs=[
            pl.BlockSpec((BQ, e), lambda i: (i, 0)),
            pl.BlockSpec((e, 3 * e), lambda i: (0, 0)),
            pl.BlockSpec((1, 3 * e), lambda i: (0, 0)),
        ],
        out_specs=pl.BlockSpec((BQ, 3 * e), lambda i: (i, 0)),
        out_shape=jax.ShapeDtypeStruct((s, 3 * e), jnp.bfloat16),
    )(x2, Wqkv, bqkv)

    bar = bar_ids.reshape(s)
    bar_c = bar_ids.reshape(s, 1)
    bar_r = bar_ids.reshape(1, s)
    inst_c = instrument_ids.reshape(s, 1)
    inst_r = instrument_ids.reshape(1, s)

    blk_last = bar[BQ - 1::BQ]
    cross_end = jnp.searchsorted(bar, blk_last, side='right').astype(jnp.int32)
    causal_end = (jnp.arange(1, nq + 1, dtype=jnp.int32)) * BQ
    kmax = jnp.maximum(cross_end, causal_end)

    attn = pl.pallas_call(
        _attn_kernel,
        grid=(nq, HEADS // 2),
        in_specs=[
            pl.BlockSpec(memory_space=pltpu.SMEM),
            pl.BlockSpec((BQ, 1), lambda i, h: (i, 0)),
            pl.BlockSpec((1, S), lambda i, h: (0, 0)),
            pl.BlockSpec((BQ, 1), lambda i, h: (i, 0)),
            pl.BlockSpec((1, S), lambda i, h: (0, 0)),
            pl.BlockSpec((S, 3 * EMBED), lambda i, h: (0, 0)),
        ],
        out_specs=pl.BlockSpec((BQ, 2 * HEAD_DIM), lambda i, h: (i, h)),
        out_shape=jax.ShapeDtypeStruct((s, e), jnp.bfloat16),
        scratch_shapes=[
            pltpu.VMEM((BQ, S), jnp.float32),
            pltpu.VMEM((BQ, S), jnp.float32),
            pltpu.VMEM((BQ, S), jnp.float32),
            pltpu.VMEM((BQ, S), jnp.float32),
            [pltpu.VMEM((BQ, 2 * HEAD_DIM), jnp.float32) for _ in range(3)],
            [pltpu.VMEM((BQ, S // BK), jnp.float32) for _ in range(6)],
        ],
    )(kmax, bar_c, bar_r, inst_c, inst_r, qkv)

    out = pl.pallas_call(
        _out_proj_kernel,
        grid=(nq,),
        in_specs=[
            pl.BlockSpec((BQ, e), lambda i: (i, 0)),
            pl.BlockSpec((e, e), lambda i: (0, 0)),
            pl.BlockSpec((1, e), lambda i: (0, 0)),
        ],
        out_specs=pl.BlockSpec((BQ, e), lambda i: (i, 0)),
        out_shape=jax.ShapeDtypeStruct((s, e), jnp.float32),
    )(attn, Wo.T.astype(jnp.bfloat16), bo.reshape(1, e))

    return out.reshape(B, s, e)


# speedup vs baseline: 1.0906x; 1.0200x over previous
import functools
import math

import jax
import jax.numpy as jnp
from jax.experimental import pallas as pl
from jax.experimental.pallas import tpu as pltpu

EMBED = 1024
HEADS = 16
HEAD_DIM = 64
SCALE = HEAD_DIM ** -0.5
WINDOW = 2
FAR = 4
S = 2048
BQ = 256
EXTB = 256

_LOG1E4 = math.log(10000.0)


def _qkv_rope_kernel(x_ref, w_ref, b_ref, o_ref):
    qi = pl.program_id(0)
    y = jnp.dot(x_ref[...], w_ref[...], preferred_element_type=jnp.float32)
    y = y + b_ref[...]
    bq, n = y.shape
    y_p32 = jnp.concatenate([y[:, 32:], y[:, :32]], axis=1)
    y_m32 = jnp.concatenate([y[:, -32:], y[:, :-32]], axis=1)
    col = jax.lax.broadcasted_iota(jnp.int32, (bq, n), 1)
    d = col % HEAD_DIM
    dr = d % (HEAD_DIM // 2)
    hi = d >= (HEAD_DIM // 2)
    partner = jnp.where(hi, y_m32, y_p32)
    inv = jnp.exp(dr.astype(jnp.float32) * (-_LOG1E4 / (HEAD_DIM // 2)))
    row = jax.lax.broadcasted_iota(jnp.int32, (bq, n), 0)
    pos = (qi * bq + row).astype(jnp.float32)
    ang = pos * inv
    c = jnp.cos(ang)
    s = jnp.sin(ang)
    roped = y * c + partner * jnp.where(hi, s, -s)
    out = jnp.where(col < 2 * EMBED, roped, y)
    out = out * jnp.where(col < EMBED, SCALE, 1.0)
    o_ref[...] = out.astype(jnp.bfloat16)


def _mask_block(qi, j0, bq, bk, bar_q, inst_q, bar_k, inst_k):
    i = qi * BQ + jax.lax.broadcasted_iota(jnp.int32, (bq, bk), 0)
    j = j0 + jax.lax.broadcasted_iota(jnp.int32, (bq, bk), 1)
    causal = j <= i
    same = (inst_q == inst_k) & (inst_q < 129) & causal
    off = bar_q - bar_k
    nearfar = ((off >= 0) & (off <= WINDOW)) | (off == FAR)
    cross = ((inst_q < 129) & (bar_q >= 0) & (inst_k != inst_q)
             & (inst_k < 129) & nearfar)
    glob = ((inst_k == 129) | (bar_k == -1)) & causal
    return same, cross, glob


def _attn_qi_kernel(cend_ref, barc_ref, barr_ref, instc_ref, instr_ref,
                    qkv_ref, o_ref, bias_ref, ms_ref, mc_ref, mg_ref,
                    acc2e_ref, s2e_ref, *, qi, kq):
    hp = pl.program_id(0)
    bar_q = barc_ref[...]
    inst_q = instc_ref[...]

    @pl.when(hp == 0)
    def _build_masks():
        same, cross, glob = _mask_block(
            qi, 0, BQ, kq, bar_q, inst_q,
            barr_ref[0:1, 0:kq], instr_ref[0:1, 0:kq])
        union = same | cross | glob
        ms_ref[...] = same.astype(jnp.float32)
        mc_ref[...] = cross.astype(jnp.float32)
        mg_ref[...] = glob.astype(jnp.float32)
        bias_ref[...] = jnp.where(union, 0.0, -1e30)

    q2 = qkv_ref[pl.ds(qi * BQ, BQ), pl.ds(hp * 128, 128)]
    kf = qkv_ref[0:kq, pl.ds(EMBED + hp * 128, 128)]
    vf = qkv_ref[0:kq, pl.ds(2 * EMBED + hp * 128, 128)]
    bias = bias_ref[...]
    ms = ms_ref[...]
    mc = mc_ref[...]
    mg = mg_ref[...]

    cend = cend_ref[qi]
    n_ext = (S - kq) // EXTB
    halves = []
    for t in range(2):
        q = q2[:, t * HEAD_DIM:(t + 1) * HEAD_DIM]
        k = kf[:, t * HEAD_DIM:(t + 1) * HEAD_DIM]
        v = vf[:, t * HEAD_DIM:(t + 1) * HEAD_DIM]
        scores = jax.lax.dot_general(
            q, k, (((1,), (1,)), ((), ())),
            preferred_element_type=jnp.float32) + bias
        m = jnp.max(scores, axis=-1, keepdims=True)

        if n_ext > 0:
            acc2e_ref[...] = jnp.zeros_like(acc2e_ref)
            s2e_ref[...] = jnp.zeros_like(s2e_ref)
            for b in range(n_ext):
                j0 = kq + b * EXTB

                @pl.when(j0 < cend)
                def _ext(j0=j0):
                    ke = qkv_ref[j0:j0 + EXTB,
                                 pl.ds(EMBED + hp * 128, 128)][
                                     :, t * HEAD_DIM:(t + 1) * HEAD_DIM]
                    ve = qkv_ref[j0:j0 + EXTB,
                                 pl.ds(2 * EMBED + hp * 128, 128)][
                                     :, t * HEAD_DIM:(t + 1) * HEAD_DIM]
                    _, cr, _ = _mask_block(
                        qi, j0, BQ, EXTB, bar_q, inst_q,
                        barr_ref[0:1, j0:j0 + EXTB],
                        instr_ref[0:1, j0:j0 + EXTB])
                    sce = jax.lax.dot_general(
                        q, ke, (((1,), (1,)), ((), ())),
                        preferred_element_type=jnp.float32)
                    ee = jnp.exp(jnp.minimum(sce - m, 80.0))
                    ee = ee * cr.astype(jnp.float32)
                    s2e_ref[...] = s2e_ref[...] + jnp.sum(
                        ee, axis=1, keepdims=True)
                    acc2e_ref[...] = acc2e_ref[...] + jnp.dot(
                        ee.astype(jnp.bfloat16), ve,
                        preferred_element_type=jnp.float32)

        e = jnp.exp(scores - m)
        s1 = jnp.sum(e * ms, axis=-1, keepdims=True)
        s2 = jnp.sum(e * mc, axis=-1, keepdims=True)
        s3 = jnp.sum(e * mg, axis=-1, keepdims=True)
        if n_ext > 0:
            s2 = s2 + s2e_ref[...]
        inv1 = 1.0 / jnp.where(s1 == 0.0, 1.0, s1)
        inv2 = 1.0 / jnp.where(s2 == 0.0, 1.0, s2)
        inv3 = 1.0 / jnp.where(s3 == 0.0, 1.0, s3)
        denom = ms * inv1 + mc * inv2 + mg * inv3
        w = (e * denom).astype(jnp.bfloat16)
        out = jnp.dot(w, v, preferred_element_type=jnp.float32)
        if n_ext > 0:
            out = out + acc2e_ref[...] * inv2
        halves.append(out)
    o_ref[...] = jnp.concatenate(halves, axis=1).astype(jnp.bfloat16)


def _out_proj_kernel(a_ref, w_ref, b_ref, o_ref):
    o_ref[...] = jnp.dot(a_ref[...], w_ref[...],
                         preferred_element_type=jnp.float32) + b_ref[...]


@jax.jit
def kernel(x, bar_ids, instrument_ids, Wq, bq, Wk, bk, Wv, bv, Wo, bo):
    B, s, e = x.shape
    x2 = x.reshape(s, e).astype(jnp.bfloat16)
    Wqkv = jnp.concatenate([Wq.T, Wk.T, Wv.T], axis=1).astype(jnp.bfloat16)
    bqkv = jnp.concatenate([bq, bk, bv]).reshape(1, 3 * e)

    nq = s // BQ
    qkv = pl.pallas_call(
        _qkv_rope_kernel,
        grid=(nq,),
        in_specs=[
            pl.BlockSpec((BQ, e), lambda i: (i, 0)),
            pl.BlockSpec((e, 3 * e), lambda i: (0, 0)),
            pl.BlockSpec((1, 3 * e), lambda i: (0, 0)),
        ],
        out_specs=pl.BlockSpec((BQ, 3 * e), lambda i: (i, 0)),
        out_shape=jax.ShapeDtypeStruct((s, 3 * e), jnp.bfloat16),
    )(x2, Wqkv, bqkv)

    bar = bar_ids.reshape(s)
    bar_c = bar_ids.reshape(s, 1)
    bar_r = bar_ids.reshape(1, s)
    inst_c = instrument_ids.reshape(s, 1)
    inst_r = instrument_ids.reshape(1, s)

    blk_last = bar[BQ - 1::BQ]
    cross_end = jnp.searchsorted(bar, blk_last, side='right').astype(jnp.int32)

    parts = []
    for qi in range(nq):
        kq = (qi + 1) * BQ
        part = pl.pallas_call(
            functools.partial(_attn_qi_kernel, qi=qi, kq=kq),
            grid=(HEADS // 2,),
            in_specs=[
                pl.BlockSpec(memory_space=pltpu.SMEM),
                pl.BlockSpec((BQ, 1), lambda h, qi=qi: (qi, 0)),
                pl.BlockSpec((1, S), lambda h: (0, 0)),
                pl.BlockSpec((BQ, 1), lambda h, qi=qi: (qi, 0)),
                pl.BlockSpec((1, S), lambda h: (0, 0)),
                pl.BlockSpec((S, 3 * EMBED), lambda h: (0, 0)),
            ],
            out_specs=pl.BlockSpec((BQ, 2 * HEAD_DIM), lambda h: (0, h)),
            out_shape=jax.ShapeDtypeStruct((BQ, e), jnp.bfloat16),
            scratch_shapes=[
                pltpu.VMEM((BQ, kq), jnp.float32),
                pltpu.VMEM((BQ, kq), jnp.float32),
                pltpu.VMEM((BQ, kq), jnp.float32),
                pltpu.VMEM((BQ, kq), jnp.float32),
                pltpu.VMEM((BQ, HEAD_DIM), jnp.float32),
                pltpu.VMEM((BQ, 1), jnp.float32),
            ],
        )(cross_end, bar_c, bar_r, inst_c, inst_r, qkv)
        parts.append(part)
    attn = jnp.concatenate(parts, axis=0)

    out = pl.pallas_call(
        _out_proj_kernel,
        grid=(nq,),
        in_specs=[
            pl.BlockSpec((BQ, e), lambda i: (i, 0)),
            pl.BlockSpec((e, e), lambda i: (0, 0)),
            pl.BlockSpec((1, e), lambda i: (0, 0)),
        ],
        out_specs=pl.BlockSpec((BQ, e), lambda i: (i, 0)),
        out_shape=jax.ShapeDtypeStruct((s, e), jnp.float32),
    )(attn, Wo.T.astype(jnp.bfloat16), bo.reshape(1, e))

    return out.reshape(B, s, e)


# speedup vs baseline: 1.2991x; 1.1912x over previous
import functools
import math

import jax
import jax.numpy as jnp
from jax.experimental import pallas as pl
from jax.experimental.pallas import tpu as pltpu

EMBED = 1024
HEADS = 16
HEAD_DIM = 64
SCALE = HEAD_DIM ** -0.5
WINDOW = 2
FAR = 4
S = 2048
BQ = 256
EXTB = 256

_LOG1E4 = math.log(10000.0)


def _qkv_rope_kernel(x_ref, w_ref, b_ref, o_ref):
    qi = pl.program_id(0)
    y = jnp.dot(x_ref[...], w_ref[...], preferred_element_type=jnp.float32)
    y = y + b_ref[...]
    bq, n = y.shape
    y_p32 = jnp.concatenate([y[:, 32:], y[:, :32]], axis=1)
    y_m32 = jnp.concatenate([y[:, -32:], y[:, :-32]], axis=1)
    col = jax.lax.broadcasted_iota(jnp.int32, (bq, n), 1)
    d = col % HEAD_DIM
    dr = d % (HEAD_DIM // 2)
    hi = d >= (HEAD_DIM // 2)
    partner = jnp.where(hi, y_m32, y_p32)
    inv = jnp.exp(dr.astype(jnp.float32) * (-_LOG1E4 / (HEAD_DIM // 2)))
    row = jax.lax.broadcasted_iota(jnp.int32, (bq, n), 0)
    pos = (qi * bq + row).astype(jnp.float32)
    ang = pos * inv
    c = jnp.cos(ang)
    s = jnp.sin(ang)
    roped = y * c + partner * jnp.where(hi, s, -s)
    out = jnp.where(col < 2 * EMBED, roped, y)
    out = out * jnp.where(col < EMBED, SCALE, 1.0)
    o_ref[...] = out.astype(jnp.bfloat16)


def _mask_block(i0, j0, bq, bk, bar_q, inst_q, bar_k, inst_k):
    i = i0 + jax.lax.broadcasted_iota(jnp.int32, (bq, bk), 0)
    j = j0 + jax.lax.broadcasted_iota(jnp.int32, (bq, bk), 1)
    causal = j <= i
    same = (inst_q == inst_k) & (inst_q < 129) & causal
    off = bar_q - bar_k
    nearfar = ((off >= 0) & (off <= WINDOW)) | (off == FAR)
    cross = ((inst_q < 129) & (bar_q >= 0) & (inst_k != inst_q)
             & (inst_k < 129) & nearfar)
    glob = ((inst_k == 129) | (bar_k == -1)) & causal
    return same, cross, glob


def _attn_variant(kq, qi, hp, cend, barc_ref, barr_ref, instc_ref, instr_ref,
                  qkv_ref, o_ref, bias_ref, ms_ref, mc_ref, mg_ref,
                  acc2e_ref, s2e_ref):
    bar_q = barc_ref[...]
    inst_q = instc_ref[...]

    @pl.when(hp == 0)
    def _build_masks():
        same, cross, glob = _mask_block(
            qi * BQ, 0, BQ, kq, bar_q, inst_q,
            barr_ref[0:1, 0:kq], instr_ref[0:1, 0:kq])
        union = same | cross | glob
        ms_ref[:, 0:kq] = same.astype(jnp.float32)
        mc_ref[:, 0:kq] = cross.astype(jnp.float32)
        mg_ref[:, 0:kq] = glob.astype(jnp.float32)
        bias_ref[:, 0:kq] = jnp.where(union, 0.0, -1e30)

    q2 = qkv_ref[pl.ds(qi * BQ, BQ), pl.ds(hp * 128, 128)]
    kf = qkv_ref[0:kq, pl.ds(EMBED + hp * 128, 128)]
    vf = qkv_ref[0:kq, pl.ds(2 * EMBED + hp * 128, 128)]
    bias = bias_ref[:, 0:kq]
    ms = ms_ref[:, 0:kq]
    mc = mc_ref[:, 0:kq]
    mg = mg_ref[:, 0:kq]

    n_ext = (S - kq) // EXTB
    halves = []
    for t in range(2):
        q = q2[:, t * HEAD_DIM:(t + 1) * HEAD_DIM]
        k = kf[:, t * HEAD_DIM:(t + 1) * HEAD_DIM]
        v = vf[:, t * HEAD_DIM:(t + 1) * HEAD_DIM]
        scores = jax.lax.dot_general(
            q, k, (((1,), (1,)), ((), ())),
            preferred_element_type=jnp.float32) + bias
        m = jnp.max(scores, axis=-1, keepdims=True)

        if n_ext > 0:
            acc2e_ref[...] = jnp.zeros_like(acc2e_ref)
            s2e_ref[...] = jnp.zeros_like(s2e_ref)
            for b in range(n_ext):
                j0 = kq + b * EXTB

                @pl.when(j0 < cend)
                def _ext(j0=j0):
                    ke = qkv_ref[j0:j0 + EXTB,
                                 pl.ds(EMBED + hp * 128, 128)][
                                     :, t * HEAD_DIM:(t + 1) * HEAD_DIM]
                    ve = qkv_ref[j0:j0 + EXTB,
                                 pl.ds(2 * EMBED + hp * 128, 128)][
                                     :, t * HEAD_DIM:(t + 1) * HEAD_DIM]
                    _, cr, _ = _mask_block(
                        qi * BQ, j0, BQ, EXTB, bar_q, inst_q,
                        barr_ref[0:1, j0:j0 + EXTB],
                        instr_ref[0:1, j0:j0 + EXTB])
                    sce = jax.lax.dot_general(
                        q, ke, (((1,), (1,)), ((), ())),
                        preferred_element_type=jnp.float32)
                    ee = jnp.exp(jnp.minimum(sce - m, 80.0))
                    ee = ee * cr.astype(jnp.float32)
                    s2e_ref[...] = s2e_ref[...] + jnp.sum(
                        ee, axis=1, keepdims=True)
                    acc2e_ref[...] = acc2e_ref[...] + jnp.dot(
                        ee.astype(jnp.bfloat16), ve,
                        preferred_element_type=jnp.float32)

        e = jnp.exp(scores - m)
        s1 = jnp.sum(e * ms, axis=-1, keepdims=True)
        s2 = jnp.sum(e * mc, axis=-1, keepdims=True)
        s3 = jnp.sum(e * mg, axis=-1, keepdims=True)
        if n_ext > 0:
            s2 = s2 + s2e_ref[...]
        inv1 = 1.0 / jnp.where(s1 == 0.0, 1.0, s1)
        inv2 = 1.0 / jnp.where(s2 == 0.0, 1.0, s2)
        inv3 = 1.0 / jnp.where(s3 == 0.0, 1.0, s3)
        denom = ms * inv1 + mc * inv2 + mg * inv3
        w = (e * denom).astype(jnp.bfloat16)
        out = jnp.dot(w, v, preferred_element_type=jnp.float32)
        if n_ext > 0:
            out = out + acc2e_ref[...] * inv2
        halves.append(out)
    o_ref[...] = jnp.concatenate(halves, axis=1).astype(jnp.bfloat16)


_EXTENTS = (1024, 2048)


def _attn_kernel(cend_ref, barc_ref, barr_ref, instc_ref, instr_ref,
                 qkv_ref, o_ref, bias_ref, ms_ref, mc_ref, mg_ref,
                 acc2e_ref, s2e_ref):
    qi = pl.program_id(0)
    hp = pl.program_id(1)
    cend = cend_ref[qi]
    args = (barc_ref, barr_ref, instc_ref, instr_ref, qkv_ref, o_ref,
            bias_ref, ms_ref, mc_ref, mg_ref, acc2e_ref, s2e_ref)

    @pl.when(qi < 4)
    def _small():
        _attn_variant(_EXTENTS[0], qi, hp, cend, *args)

    @pl.when(qi >= 4)
    def _large():
        _attn_variant(_EXTENTS[1], qi, hp, cend, *args)


def _out_proj_kernel(a_ref, w_ref, b_ref, o_ref):
    o_ref[...] = jnp.dot(a_ref[...], w_ref[...],
                         preferred_element_type=jnp.float32) + b_ref[...]


@jax.jit
def kernel(x, bar_ids, instrument_ids, Wq, bq, Wk, bk, Wv, bv, Wo, bo):
    B, s, e = x.shape
    x2 = x.reshape(s, e).astype(jnp.bfloat16)
    Wqkv = jnp.concatenate([Wq.T, Wk.T, Wv.T], axis=1).astype(jnp.bfloat16)
    bqkv = jnp.concatenate([bq, bk, bv]).reshape(1, 3 * e)

    nq = s // BQ
    qkv = pl.pallas_call(
        _qkv_rope_kernel,
        grid=(nq,),
        in_specs=[
            pl.BlockSpec((BQ, e), lambda i: (i, 0)),
            pl.BlockSpec((e, 3 * e), lambda i: (0, 0)),
            pl.BlockSpec((1, 3 * e), lambda i: (0, 0)),
        ],
        out_specs=pl.BlockSpec((BQ, 3 * e), lambda i: (i, 0)),
        out_shape=jax.ShapeDtypeStruct((s, 3 * e), jnp.bfloat16),
    )(x2, Wqkv, bqkv)

    bar = bar_ids.reshape(s)
    bar_c = bar_ids.reshape(s, 1)
    bar_r = bar_ids.reshape(1, s)
    inst_c = instrument_ids.reshape(s, 1)
    inst_r = instrument_ids.reshape(1, s)

    blk_last = bar[BQ - 1::BQ]
    cross_end = jnp.searchsorted(bar, blk_last, side='right').astype(jnp.int32)

    attn = pl.pallas_call(
        _attn_kernel,
        grid=(nq, HEADS // 2),
        in_specs=[
            pl.BlockSpec(memory_space=pltpu.SMEM),
            pl.BlockSpec((BQ, 1), lambda i, h: (i, 0)),
            pl.BlockSpec((1, S), lambda i, h: (0, 0)),
            pl.BlockSpec((BQ, 1), lambda i, h: (i, 0)),
            pl.BlockSpec((1, S), lambda i, h: (0, 0)),
            pl.BlockSpec((S, 3 * EMBED), lambda i, h: (0, 0)),
        ],
        out_specs=pl.BlockSpec((BQ, 2 * HEAD_DIM), lambda i, h: (i, h)),
        out_shape=jax.ShapeDtypeStruct((s, e), jnp.bfloat16),
        scratch_shapes=[
            pltpu.VMEM((BQ, S), jnp.float32),
            pltpu.VMEM((BQ, S), jnp.float32),
            pltpu.VMEM((BQ, S), jnp.float32),
            pltpu.VMEM((BQ, S), jnp.float32),
            pltpu.VMEM((BQ, HEAD_DIM), jnp.float32),
            pltpu.VMEM((BQ, 1), jnp.float32),
        ],
    )(cross_end, bar_c, bar_r, inst_c, inst_r, qkv)

    out = pl.pallas_call(
        _out_proj_kernel,
        grid=(nq,),
        in_specs=[
            pl.BlockSpec((BQ, e), lambda i: (i, 0)),
            pl.BlockSpec((e, e), lambda i: (0, 0)),
            pl.BlockSpec((1, e), lambda i: (0, 0)),
        ],
        out_specs=pl.BlockSpec((BQ, e), lambda i: (i, 0)),
        out_shape=jax.ShapeDtypeStruct((s, e), jnp.float32),
    )(attn, Wo.T.astype(jnp.bfloat16), bo.reshape(1, e))

    return out.reshape(B, s, e)


# speedup vs baseline: 1.7159x; 1.3209x over previous
import functools
import math

import jax
import jax.numpy as jnp
from jax.experimental import pallas as pl
from jax.experimental.pallas import tpu as pltpu

EMBED = 1024
HEADS = 16
HEAD_DIM = 64
SCALE = HEAD_DIM ** -0.5
WINDOW = 2
FAR = 4
S = 2048
BQ = 256
EXTB = 256

_LOG1E4 = math.log(10000.0)


def _qkv_rope_kernel(x_ref, wq_ref, wk_ref, wv_ref, b_ref, o_ref):
    qi = pl.program_id(0)
    y = jnp.concatenate([
        jnp.dot(x_ref[...], wq_ref[...], preferred_element_type=jnp.float32),
        jnp.dot(x_ref[...], wk_ref[...], preferred_element_type=jnp.float32),
        jnp.dot(x_ref[...], wv_ref[...], preferred_element_type=jnp.float32),
    ], axis=1)
    y = y + b_ref[...]
    bq, n = y.shape
    col = jax.lax.broadcasted_iota(jnp.int32, (bq, 128), 1)
    d = col % HEAD_DIM
    dr = d % (HEAD_DIM // 2)
    hi = d >= (HEAD_DIM // 2)
    inv = jnp.exp(dr.astype(jnp.float32) * (-_LOG1E4 / (HEAD_DIM // 2)))
    row = jax.lax.broadcasted_iota(jnp.int32, (bq, 128), 0)
    pos = (qi * bq + row).astype(jnp.float32)
    ang = pos * inv
    c = jnp.cos(ang)
    sg = jnp.sin(ang)
    sg = jnp.where(hi, sg, -sg)
    ones = jnp.ones_like(c)
    zeros = jnp.zeros_like(c)
    cfull = jnp.concatenate([c * SCALE] * 8 + [c] * 8 + [ones] * 8, axis=1)
    sfull = jnp.concatenate([sg * SCALE] * 8 + [sg] * 8 + [zeros] * 8, axis=1)
    y_p32 = jnp.concatenate([y[:, 32:], y[:, :32]], axis=1)
    y_m32 = jnp.concatenate([y[:, -32:], y[:, :-32]], axis=1)
    hi_full = jnp.concatenate([hi] * 24, axis=1)
    partner = jnp.where(hi_full, y_m32, y_p32)
    o_ref[...] = (y * cfull + partner * sfull).astype(jnp.bfloat16)


def _mask_block(i0, j0, bq, bk, bar_q, inst_q, bar_k, inst_k):
    i = i0 + jax.lax.broadcasted_iota(jnp.int32, (bq, bk), 0)
    j = j0 + jax.lax.broadcasted_iota(jnp.int32, (bq, bk), 1)
    causal = j <= i
    same = (inst_q == inst_k) & (inst_q < 129) & causal
    off = bar_q - bar_k
    nearfar = ((off >= 0) & (off <= WINDOW)) | (off == FAR)
    cross = ((inst_q < 129) & (bar_q >= 0) & (inst_k != inst_q)
             & (inst_k < 129) & nearfar)
    glob = ((inst_k == 129) | (bar_k == -1)) & causal
    return same, cross, glob


def _attn_variant(kq, qi, hp, cend, barc_ref, barr_ref, instc_ref, instr_ref,
                  qkv_ref, o_ref, bias_ref, ms_ref, mc_ref, mg_ref,
                  acc2e_ref, s2e_ref):
    bar_q = barc_ref[...]
    inst_q = instc_ref[...]

    @pl.when(hp == 0)
    def _build_masks():
        same, cross, glob = _mask_block(
            qi * BQ, 0, BQ, kq, bar_q, inst_q,
            barr_ref[0:1, 0:kq], instr_ref[0:1, 0:kq])
        union = same | cross | glob
        ms_ref[:, 0:kq] = same.astype(jnp.bfloat16)
        mc_ref[:, 0:kq] = cross.astype(jnp.bfloat16)
        mg_ref[:, 0:kq] = glob.astype(jnp.bfloat16)
        bias_ref[:, 0:kq] = jnp.where(union, 0.0, -1e30)

    q2 = qkv_ref[pl.ds(qi * BQ, BQ), pl.ds(hp * 128, 128)]
    kf = qkv_ref[0:kq, pl.ds(EMBED + hp * 128, 128)]
    vf = qkv_ref[0:kq, pl.ds(2 * EMBED + hp * 128, 128)]
    bias = bias_ref[:, 0:kq]
    ms = ms_ref[:, 0:kq]
    mc = mc_ref[:, 0:kq]
    mg = mg_ref[:, 0:kq]

    n_ext = (S - kq) // EXTB
    halves = []
    for t in range(2):
        q = q2[:, t * HEAD_DIM:(t + 1) * HEAD_DIM]
        k = kf[:, t * HEAD_DIM:(t + 1) * HEAD_DIM]
        v = vf[:, t * HEAD_DIM:(t + 1) * HEAD_DIM]
        scores = jax.lax.dot_general(
            q, k, (((1,), (1,)), ((), ())),
            preferred_element_type=jnp.float32) + bias
        m = jnp.max(scores, axis=-1, keepdims=True)

        if n_ext > 0:
            acc2e_ref[...] = jnp.zeros_like(acc2e_ref)
            s2e_ref[...] = jnp.zeros_like(s2e_ref)
            for b in range(n_ext):
                j0 = kq + b * EXTB

                @pl.when(j0 < cend)
                def _ext(j0=j0):
                    ke = qkv_ref[j0:j0 + EXTB,
                                 pl.ds(EMBED + hp * 128, 128)][
                                     :, t * HEAD_DIM:(t + 1) * HEAD_DIM]
                    ve = qkv_ref[j0:j0 + EXTB,
                                 pl.ds(2 * EMBED + hp * 128, 128)][
                                     :, t * HEAD_DIM:(t + 1) * HEAD_DIM]
                    _, cr, _ = _mask_block(
                        qi * BQ, j0, BQ, EXTB, bar_q, inst_q,
                        barr_ref[0:1, j0:j0 + EXTB],
                        instr_ref[0:1, j0:j0 + EXTB])
                    sce = jax.lax.dot_general(
                        q, ke, (((1,), (1,)), ((), ())),
                        preferred_element_type=jnp.float32)
                    ee = jnp.exp(jnp.minimum(sce - m, 80.0))
                    ee = ee * cr.astype(jnp.float32)
                    s2e_ref[...] = s2e_ref[...] + jnp.sum(
                        ee, axis=1, keepdims=True)
                    acc2e_ref[...] = acc2e_ref[...] + jnp.dot(
                        ee.astype(jnp.bfloat16), ve,
                        preferred_element_type=jnp.float32)

        e16 = jnp.exp(scores - m).astype(jnp.bfloat16)
        out = None
        for ci, mask in enumerate((ms, mc, mg)):
            em = e16 * mask
            ssum = jnp.sum(em.astype(jnp.float32), axis=-1, keepdims=True)
            if ci == 1 and n_ext > 0:
                ssum = ssum + s2e_ref[...]
            acc = jnp.dot(em, v, preferred_element_type=jnp.float32)
            if ci == 1 and n_ext > 0:
                acc = acc + acc2e_ref[...]
            part = acc / jnp.where(ssum == 0.0, 1.0, ssum)
            out = part if out is None else out + part
        halves.append(out)
    o_ref[...] = jnp.concatenate(halves, axis=1).astype(jnp.bfloat16)


def _attn_kernel(cend_ref, barc_ref, barr_ref, instc_ref, instr_ref,
                 qkv_ref, o_ref, bias_ref, ms_ref, mc_ref, mg_ref,
                 acc2e_ref, s2e_ref):
    qi = pl.program_id(0)
    hp = pl.program_id(1)
    cend = cend_ref[qi]
    args = (barc_ref, barr_ref, instc_ref, instr_ref, qkv_ref, o_ref,
            bias_ref, ms_ref, mc_ref, mg_ref, acc2e_ref, s2e_ref)

    for g in range(4):
        @pl.when((qi >= 2 * g) & (qi < 2 * g + 2))
        def _var(g=g):
            _attn_variant(512 * (g + 1), qi, hp, cend, *args)


def _out_proj_kernel(a_ref, w_ref, b_ref, o_ref):
    o_ref[...] = jnp.dot(a_ref[...], w_ref[...],
                         preferred_element_type=jnp.float32) + b_ref[...]


@jax.jit
def kernel(x, bar_ids, instrument_ids, Wq, bq, Wk, bk, Wv, bv, Wo, bo):
    B, s, e = x.shape
    x2 = x.reshape(s, e).astype(jnp.bfloat16)
    bqkv = jnp.concatenate([bq, bk, bv]).reshape(1, 3 * e)

    nq = s // BQ
    qkv = pl.pallas_call(
        _qkv_rope_kernel,
        grid=(nq,),
        in_specs=[
            pl.BlockSpec((BQ, e), lambda i: (i, 0)),
            pl.BlockSpec((e, e), lambda i: (0, 0)),
            pl.BlockSpec((e, e), lambda i: (0, 0)),
            pl.BlockSpec((e, e), lambda i: (0, 0)),
            pl.BlockSpec((1, 3 * e), lambda i: (0, 0)),
        ],
        out_specs=pl.BlockSpec((BQ, 3 * e), lambda i: (i, 0)),
        out_shape=jax.ShapeDtypeStruct((s, 3 * e), jnp.bfloat16),
    )(x2, Wq.T.astype(jnp.bfloat16), Wk.T.astype(jnp.bfloat16),
      Wv.T.astype(jnp.bfloat16), bqkv)

    bar = bar_ids.reshape(s)
    bar_c = bar_ids.reshape(s, 1)
    bar_r = bar_ids.reshape(1, s)
    inst_c = instrument_ids.reshape(s, 1)
    inst_r = instrument_ids.reshape(1, s)

    blk_last = bar[BQ - 1::BQ]
    cross_end = jnp.sum(bar[None, :] <= blk_last[:, None],
                        axis=1).astype(jnp.int32)

    attn = pl.pallas_call(
        _attn_kernel,
        grid=(nq, HEADS // 2),
        in_specs=[
            pl.BlockSpec(memory_space=pltpu.SMEM),
            pl.BlockSpec((BQ, 1), lambda i, h: (i, 0)),
            pl.BlockSpec((1, S), lambda i, h: (0, 0)),
            pl.BlockSpec((BQ, 1), lambda i, h: (i, 0)),
            pl.BlockSpec((1, S), lambda i, h: (0, 0)),
            pl.BlockSpec((S, 3 * EMBED), lambda i, h: (0, 0)),
        ],
        out_specs=pl.BlockSpec((BQ, 2 * HEAD_DIM), lambda i, h: (i, h)),
        out_shape=jax.ShapeDtypeStruct((s, e), jnp.bfloat16),
        scratch_shapes=[
            pltpu.VMEM((BQ, S), jnp.float32),
            pltpu.VMEM((BQ, S), jnp.bfloat16),
            pltpu.VMEM((BQ, S), jnp.bfloat16),
            pltpu.VMEM((BQ, S), jnp.bfloat16),
            pltpu.VMEM((BQ, HEAD_DIM), jnp.float32),
            pltpu.VMEM((BQ, 1), jnp.float32),
        ],
    )(cross_end, bar_c, bar_r, inst_c, inst_r, qkv)

    out = pl.pallas_call(
        _out_proj_kernel,
        grid=(nq,),
        in_specs=[
            pl.BlockSpec((BQ, e), lambda i: (i, 0)),
            pl.BlockSpec((e, e), lambda i: (0, 0)),
            pl.BlockSpec((1, e), lambda i: (0, 0)),
        ],
        out_specs=pl.BlockSpec((BQ, e), lambda i: (i, 0)),
        out_shape=jax.ShapeDtypeStruct((s, e), jnp.float32),
    )(attn, Wo.T.astype(jnp.bfloat16), bo.reshape(1, e))

    return out.reshape(B, s, e)


# speedup vs baseline: 1.7661x; 1.0293x over previous
import functools
import math

import jax
import jax.numpy as jnp
from jax.experimental import pallas as pl
from jax.experimental.pallas import tpu as pltpu

EMBED = 1024
HEADS = 16
HEAD_DIM = 64
SCALE = HEAD_DIM ** -0.5
WINDOW = 2
FAR = 4
S = 2048
BQ = 256
EXTB = 256

_LOG1E4 = math.log(10000.0)


def _qkv_rope_kernel(x_ref, wq_ref, wk_ref, wv_ref, b_ref, o_ref):
    qi = pl.program_id(0)
    y = jnp.concatenate([
        jnp.dot(x_ref[...], wq_ref[...], preferred_element_type=jnp.float32),
        jnp.dot(x_ref[...], wk_ref[...], preferred_element_type=jnp.float32),
        jnp.dot(x_ref[...], wv_ref[...], preferred_element_type=jnp.float32),
    ], axis=1)
    y = y + b_ref[...]
    bq, n = y.shape
    col = jax.lax.broadcasted_iota(jnp.int32, (bq, 128), 1)
    d = col % HEAD_DIM
    dr = d % (HEAD_DIM // 2)
    hi = d >= (HEAD_DIM // 2)
    inv = jnp.exp(dr.astype(jnp.float32) * (-_LOG1E4 / (HEAD_DIM // 2)))
    row = jax.lax.broadcasted_iota(jnp.int32, (bq, 128), 0)
    pos = (qi * bq + row).astype(jnp.float32)
    ang = pos * inv
    c = jnp.cos(ang)
    sg = jnp.sin(ang)
    sg = jnp.where(hi, sg, -sg)
    ones = jnp.ones_like(c)
    zeros = jnp.zeros_like(c)
    cfull = jnp.concatenate([c * SCALE] * 8 + [c] * 8 + [ones] * 8, axis=1)
    sfull = jnp.concatenate([sg * SCALE] * 8 + [sg] * 8 + [zeros] * 8, axis=1)
    y_p32 = jnp.concatenate([y[:, 32:], y[:, :32]], axis=1)
    y_m32 = jnp.concatenate([y[:, -32:], y[:, :-32]], axis=1)
    hi_full = jnp.concatenate([hi] * 24, axis=1)
    partner = jnp.where(hi_full, y_m32, y_p32)
    o_ref[...] = (y * cfull + partner * sfull).astype(jnp.bfloat16)


def _mask_block(i0, j0, bq, bk, bar_q, inst_q, bar_k, inst_k):
    i = i0 + jax.lax.broadcasted_iota(jnp.int32, (bq, bk), 0)
    j = j0 + jax.lax.broadcasted_iota(jnp.int32, (bq, bk), 1)
    causal = j <= i
    same = (inst_q == inst_k) & (inst_q < 129) & causal
    off = bar_q - bar_k
    nearfar = ((off >= 0) & (off <= WINDOW)) | (off == FAR)
    cross = ((inst_q < 129) & (bar_q >= 0) & (inst_k != inst_q)
             & (inst_k < 129) & nearfar)
    glob = ((inst_k == 129) | (bar_k == -1)) & causal
    return same, cross, glob


def _attn_variant(kq, qi, hp, cend, barc_ref, barr_ref, instc_ref, instr_ref,
                  qkv_ref, o_ref, ms_ref, mc_ref, mg_ref,
                  acc2e_ref, s2e_ref):
    bar_q = barc_ref[...]
    inst_q = instc_ref[...]

    @pl.when(hp == 0)
    def _build_masks():
        same, cross, glob = _mask_block(
            qi * BQ, 0, BQ, kq, bar_q, inst_q,
            barr_ref[0:1, 0:kq], instr_ref[0:1, 0:kq])
        ms_ref[:, 0:kq] = same.astype(jnp.bfloat16)
        mc_ref[:, 0:kq] = cross.astype(jnp.bfloat16)
        mg_ref[:, 0:kq] = glob.astype(jnp.bfloat16)

    q2 = qkv_ref[pl.ds(qi * BQ, BQ), pl.ds(hp * 128, 128)]
    kf = qkv_ref[0:kq, pl.ds(EMBED + hp * 128, 128)]
    vf = qkv_ref[0:kq, pl.ds(2 * EMBED + hp * 128, 128)]
    ms = ms_ref[:, 0:kq]
    mc = mc_ref[:, 0:kq]
    mg = mg_ref[:, 0:kq]

    n_ext = (S - kq) // EXTB
    halves = []
    for t in range(2):
        q = q2[:, t * HEAD_DIM:(t + 1) * HEAD_DIM]
        k = kf[:, t * HEAD_DIM:(t + 1) * HEAD_DIM]
        v = vf[:, t * HEAD_DIM:(t + 1) * HEAD_DIM]
        scores = jax.lax.dot_general(
            q, k, (((1,), (1,)), ((), ())),
            preferred_element_type=jnp.float32)
        m = jnp.max(scores, axis=-1, keepdims=True)

        if n_ext > 0:
            acc2e_ref[...] = jnp.zeros_like(acc2e_ref)
            s2e_ref[...] = jnp.zeros_like(s2e_ref)
            for b in range(n_ext):
                j0 = kq + b * EXTB

                @pl.when(j0 < cend)
                def _ext(j0=j0):
                    ke = qkv_ref[j0:j0 + EXTB,
                                 pl.ds(EMBED + hp * 128, 128)][
                                     :, t * HEAD_DIM:(t + 1) * HEAD_DIM]
                    ve = qkv_ref[j0:j0 + EXTB,
                                 pl.ds(2 * EMBED + hp * 128, 128)][
                                     :, t * HEAD_DIM:(t + 1) * HEAD_DIM]
                    _, cr, _ = _mask_block(
                        qi * BQ, j0, BQ, EXTB, bar_q, inst_q,
                        barr_ref[0:1, j0:j0 + EXTB],
                        instr_ref[0:1, j0:j0 + EXTB])
                    sce = jax.lax.dot_general(
                        q, ke, (((1,), (1,)), ((), ())),
                        preferred_element_type=jnp.float32)
                    ee = jnp.exp(jnp.minimum(sce - m, 80.0))
                    ee = ee * cr.astype(jnp.float32)
                    s2e_ref[...] = s2e_ref[...] + jnp.sum(
                        ee, axis=1, keepdims=True)
                    acc2e_ref[...] = acc2e_ref[...] + jnp.dot(
                        ee.astype(jnp.bfloat16), ve,
                        preferred_element_type=jnp.float32)

        e16 = jnp.exp(scores - m).astype(jnp.bfloat16)
        out = None
        for ci, mask in enumerate((ms, mc, mg)):
            em = e16 * mask
            ssum = jnp.sum(em.astype(jnp.float32), axis=-1, keepdims=True)
            if ci == 1 and n_ext > 0:
                ssum = ssum + s2e_ref[...]
            acc = jnp.dot(em, v, preferred_element_type=jnp.float32)
            if ci == 1 and n_ext > 0:
                acc = acc + acc2e_ref[...]
            part = acc / jnp.where(ssum == 0.0, 1.0, ssum)
            out = part if out is None else out + part
        halves.append(out)
    o_ref[...] = jnp.concatenate(halves, axis=1).astype(jnp.bfloat16)


def _attn_kernel(cend_ref, barc_ref, barr_ref, instc_ref, instr_ref,
                 qkv_ref, o_ref, ms_ref, mc_ref, mg_ref,
                 acc2e_ref, s2e_ref):
    qi = pl.program_id(0)
    hp = pl.program_id(1)
    cend = cend_ref[qi]
    args = (barc_ref, barr_ref, instc_ref, instr_ref, qkv_ref, o_ref,
            ms_ref, mc_ref, mg_ref, acc2e_ref, s2e_ref)

    for g in range(4):
        @pl.when((qi >= 2 * g) & (qi < 2 * g + 2))
        def _var(g=g):
            _attn_variant(512 * (g + 1), qi, hp, cend, *args)


def _out_proj_kernel(a_ref, w_ref, b_ref, o_ref):
    o_ref[...] = jnp.dot(a_ref[...], w_ref[...],
                         preferred_element_type=jnp.float32) + b_ref[...]


@jax.jit
def kernel(x, bar_ids, instrument_ids, Wq, bq, Wk, bk, Wv, bv, Wo, bo):
    B, s, e = x.shape
    x2 = x.reshape(s, e).astype(jnp.bfloat16)
    bqkv = jnp.concatenate([bq, bk, bv]).reshape(1, 3 * e)

    nq = s // BQ
    qkv = pl.pallas_call(
        _qkv_rope_kernel,
        grid=(nq,),
        in_specs=[
            pl.BlockSpec((BQ, e), lambda i: (i, 0)),
            pl.BlockSpec((e, e), lambda i: (0, 0)),
            pl.BlockSpec((e, e), lambda i: (0, 0)),
            pl.BlockSpec((e, e), lambda i: (0, 0)),
            pl.BlockSpec((1, 3 * e), lambda i: (0, 0)),
        ],
        out_specs=pl.BlockSpec((BQ, 3 * e), lambda i: (i, 0)),
        out_shape=jax.ShapeDtypeStruct((s, 3 * e), jnp.bfloat16),
    )(x2, Wq.T.astype(jnp.bfloat16), Wk.T.astype(jnp.bfloat16),
      Wv.T.astype(jnp.bfloat16), bqkv)

    bar = bar_ids.reshape(s)
    bar_c = bar_ids.reshape(s, 1)
    bar_r = bar_ids.reshape(1, s)
    inst_c = instrument_ids.reshape(s, 1)
    inst_r = instrument_ids.reshape(1, s)

    blk_last = bar[BQ - 1::BQ]
    cross_end = jnp.sum(bar[None, :] <= blk_last[:, None],
                        axis=1).astype(jnp.int32)

    attn = pl.pallas_call(
        _attn_kernel,
        grid=(nq, HEADS // 2),
        in_specs=[
            pl.BlockSpec(memory_space=pltpu.SMEM),
            pl.BlockSpec((BQ, 1), lambda i, h: (i, 0)),
            pl.BlockSpec((1, S), lambda i, h: (0, 0)),
            pl.BlockSpec((BQ, 1), lambda i, h: (i, 0)),
            pl.BlockSpec((1, S), lambda i, h: (0, 0)),
            pl.BlockSpec((S, 3 * EMBED), lambda i, h: (0, 0)),
        ],
        out_specs=pl.BlockSpec((BQ, 2 * HEAD_DIM), lambda i, h: (i, h)),
        out_shape=jax.ShapeDtypeStruct((s, e), jnp.bfloat16),
        scratch_shapes=[
            pltpu.VMEM((BQ, S), jnp.bfloat16),
            pltpu.VMEM((BQ, S), jnp.bfloat16),
            pltpu.VMEM((BQ, S), jnp.bfloat16),
            pltpu.VMEM((BQ, HEAD_DIM), jnp.float32),
            pltpu.VMEM((BQ, 1), jnp.float32),
        ],
    )(cross_end, bar_c, bar_r, inst_c, inst_r, qkv)

    out = pl.pallas_call(
        _out_proj_kernel,
        grid=(nq,),
        in_specs=[
            pl.BlockSpec((BQ, e), lambda i: (i, 0)),
            pl.BlockSpec((e, e), lambda i: (0, 0)),
            pl.BlockSpec((1, e), lambda i: (0, 0)),
        ],
        out_specs=pl.BlockSpec((BQ, e), lambda i: (i, 0)),
        out_shape=jax.ShapeDtypeStruct((s, e), jnp.float32),
    )(attn, Wo.T.astype(jnp.bfloat16), bo.reshape(1, e))

    return out.reshape(B, s, e)
